# baseline (speedup 1.0000x reference)
import functools

import jax
import jax.numpy as jnp
from jax import lax
from jax.experimental import pallas as pl
from jax.experimental.pallas import tpu as pltpu
from jax.experimental.pallas import tpu_sc as plsc

B = 1024
E = 32
V = 100000

_NC = 2
_NS = 16
_NW = _NC * _NS
_B_PER_W = B // _NW

_V_ALIGNED = (V // 128) * 128
_C_MAX = _V_ALIGNED - 128
_TAIL = V - _V_ALIGNED
_NSLAB = 16


def _gather_body(tt_hbm, idx_hbm, tail_hbm, out_hbm,
                 idx_v, tail_v, ebuf, slabs, sems, tsem):
    wid = lax.axis_index("s") * _NC + lax.axis_index("c")
    base = wid * _B_PER_W
    pltpu.sync_copy(idx_hbm.at[wid], idx_v)
    pltpu.async_copy(tail_hbm, tail_v, tsem).wait()
    vecs = [idx_v[pl.ds(0, 16)], idx_v[pl.ds(16, 16)]]

    def v_of(j):
        return vecs[j // 16][j % 16]

    def fire(j):
        v = v_of(j)
        c = pl.multiple_of(jnp.minimum((v >> 7) * 128, _C_MAX), 128)
        return pltpu.async_copy(
            tt_hbm.at[:, pl.ds(c, 128)], slabs.at[j % _NSLAB], sems.at[j % _NSLAB])

    e_half = lax.iota(jnp.int32, 16)
    descs = {}
    for j in range(_NSLAB):
        descs[j] = fire(j)
    for j in range(_B_PER_W):
        descs[j].wait()
        v = v_of(j)
        c = jnp.minimum((v >> 7) * 128, _C_MAX)
        col = jnp.minimum(v - c, 127)
        t_row = jnp.clip(v - _V_ALIGNED, 0, _TAIL - 1)
        is_tail = jnp.full((16,), 0, jnp.int32) + (v >= _V_ALIGNED).astype(jnp.int32)
        j_splat = jnp.full((16,), j, jnp.int32)
        for h in range(E // 16):
            e_idx = e_half + (16 * h)
            main = plsc.load_gather(
                slabs.at[j % _NSLAB], [e_idx, jnp.full((16,), 0, jnp.int32) + col])
            tail = plsc.load_gather(
                tail_v, [jnp.full((16,), 0, jnp.int32) + t_row, e_idx])
            vals = jnp.where(is_tail == 1, tail, main)
            plsc.store_scatter(ebuf, [j_splat, e_idx], vals)
        if j + _NSLAB < _B_PER_W:
            descs[j + _NSLAB] = fire(j + _NSLAB)
    pltpu.sync_copy(ebuf, out_hbm.at[pl.ds(base, _B_PER_W)])


_sc_gather = functools.partial(
    pl.kernel,
    mesh=plsc.VectorSubcoreMesh(core_axis_name="c", subcore_axis_name="s"),
    out_type=jax.ShapeDtypeStruct((B, E), jnp.float32),
    scratch_types=[
        pltpu.VMEM((_B_PER_W,), jnp.int32),
        pltpu.VMEM((_TAIL, E), jnp.float32),
        pltpu.VMEM((_B_PER_W, E), jnp.float32),
        pltpu.VMEM((_NSLAB, E, 128), jnp.float32),
        pltpu.SemaphoreType.DMA((_NSLAB,)),
        pltpu.SemaphoreType.DMA,
    ],
    compiler_params=pltpu.CompilerParams(
        use_tc_tiling_on_sc=True, needs_layout_passes=False),
)(_gather_body)


_V_CH = 4096
_NV = pl.cdiv(V, _V_CH)
_V_PAD = _NV * _V_CH


def _proj_body_vgrid(embt_ref, wt_ref, b_ref, out_ref):
    wt_aug = jnp.concatenate([wt_ref[...], b_ref[...]], axis=0)
    ones = jnp.ones((1, B), jnp.float32)
    embt_aug = jnp.concatenate([embt_ref[...], ones], axis=0)
    out_ref[...] = lax.dot_general(
        wt_aug, embt_aug,
        dimension_numbers=(((0,), (0,)), ((), ())),
        preferred_element_type=jnp.float32,
    )


def _tc_project_vgrid(embt, W, b):
    wt = W.T
    b2 = b.reshape(1, V)
    out_t = pl.pallas_call(
        _proj_body_vgrid,
        grid=(_NV,),
        in_specs=[
            pl.BlockSpec((E, B), lambda i: (0, 0)),
            pl.BlockSpec((E, _V_CH), lambda i: (0, i)),
            pl.BlockSpec((1, _V_CH), lambda i: (0, i)),
        ],
        out_specs=pl.BlockSpec((_V_CH, B), lambda i: (i, 0)),
        out_shape=jax.ShapeDtypeStruct((V, B), jnp.float32),
    )(embt, wt, b2)
    return out_t.T


def kernel(data, emb_table, W, b):
    tail = emb_table[_V_ALIGNED:]
    emb = _sc_gather(emb_table.T, data.reshape(_NW, _B_PER_W), tail)
    return _tc_project_vgrid(emb.T, W, b)


# speedup vs baseline: 1.0014x; 1.0014x over previous
import functools

import jax
import jax.numpy as jnp
from jax import lax
from jax.experimental import pallas as pl
from jax.experimental.pallas import tpu as pltpu
from jax.experimental.pallas import tpu_sc as plsc

B = 1024
E = 32
V = 100000

_NC = 2
_NS = 16
_NW = _NC * _NS
_B_PER_W = B // _NW

_V_ALIGNED = (V // 128) * 128
_C_MAX = _V_ALIGNED - 128
_TAIL = V - _V_ALIGNED
_NSLAB = 8


def _gather_body(tt_hbm, idx_hbm, tail_hbm, out_hbm,
                 idx_v, tail_v, ebuf, slabs, sems, tsem):
    wid = lax.axis_index("s") * _NC + lax.axis_index("c")
    base = wid * _B_PER_W
    pltpu.sync_copy(idx_hbm.at[pl.ds(base, _B_PER_W)], idx_v)
    pltpu.async_copy(tail_hbm, tail_v, tsem).wait()
    vecs = [idx_v[pl.ds(0, 16)], idx_v[pl.ds(16, 16)]]

    def v_of(j):
        return vecs[j // 16][j % 16]

    def fire(j):
        v = v_of(j)
        c = pl.multiple_of(jnp.minimum((v >> 7) * 128, _C_MAX), 128)
        return pltpu.async_copy(
            tt_hbm.at[:, pl.ds(c, 128)], slabs.at[j % _NSLAB], sems.at[j % _NSLAB])

    e_half = lax.iota(jnp.int32, 16)
    descs = {}
    for j in range(_NSLAB):
        descs[j] = fire(j)
    for j in range(_B_PER_W):
        descs[j].wait()
        v = v_of(j)
        c = jnp.minimum((v >> 7) * 128, _C_MAX)
        col = jnp.minimum(v - c, 127)
        t_row = jnp.clip(v - _V_ALIGNED, 0, _TAIL - 1)
        is_tail = jnp.full((16,), 0, jnp.int32) + (v >= _V_ALIGNED).astype(jnp.int32)
        j_splat = jnp.full((16,), j, jnp.int32)
        for h in range(E // 16):
            e_idx = e_half + (16 * h)
            main = plsc.load_gather(
                slabs.at[j % _NSLAB], [e_idx, jnp.full((16,), 0, jnp.int32) + col])
            tail = plsc.load_gather(
                tail_v, [jnp.full((16,), 0, jnp.int32) + t_row, e_idx])
            vals = jnp.where(is_tail == 1, tail, main)
            plsc.store_scatter(ebuf, [j_splat, e_idx], vals)
        if j + _NSLAB < _B_PER_W:
            descs[j + _NSLAB] = fire(j + _NSLAB)
    pltpu.sync_copy(ebuf, out_hbm.at[pl.ds(base, _B_PER_W)])


_sc_gather = functools.partial(
    pl.kernel,
    mesh=plsc.VectorSubcoreMesh(core_axis_name="c", subcore_axis_name="s"),
    out_type=jax.ShapeDtypeStruct((B, E), jnp.float32),
    scratch_types=[
        pltpu.VMEM((_B_PER_W,), jnp.int32),
        pltpu.VMEM((_TAIL, E), jnp.float32),
        pltpu.VMEM((_B_PER_W, E), jnp.float32),
        pltpu.VMEM((_NSLAB, E, 128), jnp.float32),
        pltpu.SemaphoreType.DMA((_NSLAB,)),
        pltpu.SemaphoreType.DMA,
    ],
    compiler_params=pltpu.CompilerParams(
        use_tc_tiling_on_sc=True, needs_layout_passes=False),
)(_gather_body)


_V_CH = 4096
_NV = pl.cdiv(V, _V_CH)
_V_PAD = _NV * _V_CH


def _proj_body_vgrid(embt_ref, wt_ref, b_ref, out_ref):
    wt_aug = jnp.concatenate([wt_ref[...], b_ref[...]], axis=0)
    ones = jnp.ones((1, B), jnp.float32)
    embt_aug = jnp.concatenate([embt_ref[...], ones], axis=0)
    out_ref[...] = lax.dot_general(
        wt_aug, embt_aug,
        dimension_numbers=(((0,), (0,)), ((), ())),
        preferred_element_type=jnp.float32,
    )


def _tc_project_vgrid(embt, W, b):
    wt = W.T
    b2 = b.reshape(1, V)
    out_t = pl.pallas_call(
        _proj_body_vgrid,
        grid=(_NV,),
        in_specs=[
            pl.BlockSpec((E, B), lambda i: (0, 0)),
            pl.BlockSpec((E, _V_CH), lambda i: (0, i)),
            pl.BlockSpec((1, _V_CH), lambda i: (0, i)),
        ],
        out_specs=pl.BlockSpec((_V_CH, B), lambda i: (i, 0)),
        out_shape=jax.ShapeDtypeStruct((V, B), jnp.float32),
    )(embt, wt, b2)
    return out_t.T


def kernel(data, emb_table, W, b):
    tail = emb_table[_V_ALIGNED:]
    emb = _sc_gather(emb_table.T, data, tail)
    return _tc_project_vgrid(emb.T, W, b)


# speedup vs baseline: 1.0017x; 1.0003x over previous
import functools

import jax
import jax.numpy as jnp
from jax import lax
from jax.experimental import pallas as pl
from jax.experimental.pallas import tpu as pltpu
from jax.experimental.pallas import tpu_sc as plsc

B = 1024
E = 32
V = 100000

_NC = 2
_NS = 16
_NW = _NC * _NS
_B_PER_W = B // _NW

_V_ALIGNED = (V // 128) * 128
_C_MAX = _V_ALIGNED - 128
_TAIL = V - _V_ALIGNED
_NSLAB = 8


def _gather_body(tt_hbm, idx_hbm, tail_hbm, out_hbm,
                 idx_v, tail_v, ebuf, slabs, sems, tsem):
    wid = lax.axis_index("s") * _NC + lax.axis_index("c")
    base = wid * _B_PER_W
    pltpu.sync_copy(idx_hbm.at[pl.ds(base, _B_PER_W)], idx_v)
    pltpu.async_copy(tail_hbm, tail_v, tsem).wait()
    vecs = [idx_v[pl.ds(0, 16)], idx_v[pl.ds(16, 16)]]

    def v_of(j):
        return vecs[j // 16][j % 16]

    def fire(j):
        v = v_of(j)
        c = pl.multiple_of(jnp.minimum((v >> 7) * 128, _C_MAX), 128)
        return pltpu.async_copy(
            tt_hbm.at[:, pl.ds(c, 128)], slabs.at[j % _NSLAB], sems.at[j % _NSLAB])

    e_half = lax.iota(jnp.int32, 16)
    descs = {}
    for j in range(_NSLAB):
        descs[j] = fire(j)
    for j in range(_B_PER_W):
        descs[j].wait()
        v = v_of(j)
        c = jnp.minimum((v >> 7) * 128, _C_MAX)
        col = jnp.minimum(v - c, 127)
        t_row = jnp.clip(v - _V_ALIGNED, 0, _TAIL - 1)
        is_tail = jnp.full((16,), 0, jnp.int32) + (v >= _V_ALIGNED).astype(jnp.int32)
        j_splat = jnp.full((16,), j, jnp.int32)
        for h in range(E // 16):
            e_idx = e_half + (16 * h)
            main = plsc.load_gather(
                slabs.at[j % _NSLAB], [e_idx, jnp.full((16,), 0, jnp.int32) + col])
            tail = plsc.load_gather(
                tail_v, [jnp.full((16,), 0, jnp.int32) + t_row, e_idx])
            vals = jnp.where(is_tail == 1, tail, main)
            plsc.store_scatter(ebuf, [j_splat, e_idx], vals)
        if j + _NSLAB < _B_PER_W:
            descs[j + _NSLAB] = fire(j + _NSLAB)
    pltpu.sync_copy(ebuf, out_hbm.at[pl.ds(base, _B_PER_W)])


_sc_gather = functools.partial(
    pl.kernel,
    mesh=plsc.VectorSubcoreMesh(core_axis_name="c", subcore_axis_name="s"),
    out_type=jax.ShapeDtypeStruct((B, E), jnp.float32),
    scratch_types=[
        pltpu.VMEM((_B_PER_W,), jnp.int32),
        pltpu.VMEM((_TAIL, E), jnp.float32),
        pltpu.VMEM((_B_PER_W, E), jnp.float32),
        pltpu.VMEM((_NSLAB, E, 128), jnp.float32),
        pltpu.SemaphoreType.DMA((_NSLAB,)),
        pltpu.SemaphoreType.DMA,
    ],
    compiler_params=pltpu.CompilerParams(
        use_tc_tiling_on_sc=True, needs_layout_passes=False),
)(_gather_body)


_V_CH = 5120
_NV = pl.cdiv(V, _V_CH)
_V_PAD = _NV * _V_CH


def _proj_body_vgrid(embt_ref, wt_ref, b_ref, out_ref):
    wt_aug = jnp.concatenate([wt_ref[...], b_ref[...]], axis=0)
    ones = jnp.ones((1, B), jnp.float32)
    embt_aug = jnp.concatenate([embt_ref[...], ones], axis=0)
    out_ref[...] = lax.dot_general(
        wt_aug, embt_aug,
        dimension_numbers=(((0,), (0,)), ((), ())),
        preferred_element_type=jnp.float32,
    )


def _tc_project_vgrid(embt, W, b):
    wt = W.T
    b2 = b.reshape(1, V)
    out_t = pl.pallas_call(
        _proj_body_vgrid,
        grid=(_NV,),
        in_specs=[
            pl.BlockSpec((E, B), lambda i: (0, 0)),
            pl.BlockSpec((E, _V_CH), lambda i: (0, i)),
            pl.BlockSpec((1, _V_CH), lambda i: (0, i)),
        ],
        out_specs=pl.BlockSpec((_V_CH, B), lambda i: (i, 0)),
        out_shape=jax.ShapeDtypeStruct((V, B), jnp.float32),
    )(embt, wt, b2)
    return out_t.T


def kernel(data, emb_table, W, b):
    tail = emb_table[_V_ALIGNED:]
    emb = _sc_gather(emb_table.T, data, tail)
    return _tc_project_vgrid(emb.T, W, b)


# speedup vs baseline: 1.0055x; 1.0038x over previous
import functools

import jax
import jax.numpy as jnp
from jax import lax
from jax.experimental import pallas as pl
from jax.experimental.pallas import tpu as pltpu
from jax.experimental.pallas import tpu_sc as plsc

B = 1024
E = 32
V = 100000

_NC = 2
_NS = 16
_NW = _NC * _NS
_B_PER_W = B // _NW

_V_ALIGNED = (V // 128) * 128
_C_MAX = _V_ALIGNED - 128
_TAIL = V - _V_ALIGNED
_NSLAB = 8


def _gather_body(tt_hbm, idx_hbm, tail_hbm, out_hbm,
                 idx_v, tail_v, ebuf, slabs, sems, tsem):
    wid = lax.axis_index("s") * _NC + lax.axis_index("c")
    base = wid * _B_PER_W
    pltpu.sync_copy(idx_hbm.at[pl.ds(base, _B_PER_W)], idx_v)
    pltpu.async_copy(tail_hbm, tail_v, tsem).wait()
    vecs = [idx_v[pl.ds(0, 16)], idx_v[pl.ds(16, 16)]]

    def v_of(j):
        return vecs[j // 16][j % 16]

    def fire(j):
        v = v_of(j)
        c = pl.multiple_of(jnp.minimum((v >> 7) * 128, _C_MAX), 128)
        return pltpu.async_copy(
            tt_hbm.at[:, pl.ds(c, 128)], slabs.at[j % _NSLAB], sems.at[j % _NSLAB])

    e_half = lax.iota(jnp.int32, 16)
    descs = {}
    for j in range(_NSLAB):
        descs[j] = fire(j)
    for j in range(_B_PER_W):
        descs[j].wait()
        v = v_of(j)
        c = jnp.minimum((v >> 7) * 128, _C_MAX)
        col = jnp.minimum(v - c, 127)
        t_row = jnp.clip(v - _V_ALIGNED, 0, _TAIL - 1)
        is_tail = jnp.full((16,), 0, jnp.int32) + (v >= _V_ALIGNED).astype(jnp.int32)
        j_splat = jnp.full((16,), j, jnp.int32)
        for h in range(E // 16):
            e_idx = e_half + (16 * h)
            main = plsc.load_gather(
                slabs.at[j % _NSLAB], [e_idx, jnp.full((16,), 0, jnp.int32) + col])
            tail = plsc.load_gather(
                tail_v, [jnp.full((16,), 0, jnp.int32) + t_row, e_idx])
            vals = jnp.where(is_tail == 1, tail, main)
            plsc.store_scatter(ebuf, [j_splat, e_idx], vals)
        if j + _NSLAB < _B_PER_W:
            descs[j + _NSLAB] = fire(j + _NSLAB)
    pltpu.sync_copy(ebuf, out_hbm.at[pl.ds(base, _B_PER_W)])


_sc_gather = functools.partial(
    pl.kernel,
    mesh=plsc.VectorSubcoreMesh(core_axis_name="c", subcore_axis_name="s"),
    out_type=jax.ShapeDtypeStruct((B, E), jnp.float32),
    scratch_types=[
        pltpu.VMEM((_B_PER_W,), jnp.int32),
        pltpu.VMEM((_TAIL, E), jnp.float32),
        pltpu.VMEM((_B_PER_W, E), jnp.float32),
        pltpu.VMEM((_NSLAB, E, 128), jnp.float32),
        pltpu.SemaphoreType.DMA((_NSLAB,)),
        pltpu.SemaphoreType.DMA,
    ],
    compiler_params=pltpu.CompilerParams(
        use_tc_tiling_on_sc=True, needs_layout_passes=False),
)(_gather_body)


_V_CH = 4096
_NV = pl.cdiv(V, _V_CH)
_V_PAD = _NV * _V_CH


def _proj_body_vgrid(embt_ref, wt_ref, b_ref, out_ref):
    wt_aug = jnp.concatenate([wt_ref[...], b_ref[...]], axis=0)
    ones = jnp.ones((1, B), jnp.float32)
    embt_aug = jnp.concatenate([embt_ref[...], ones], axis=0)
    out_ref[...] = lax.dot_general(
        wt_aug, embt_aug,
        dimension_numbers=(((0,), (0,)), ((), ())),
        preferred_element_type=jnp.float32,
    )


def _tc_project_vgrid(embt, W, b):
    wt = W.T
    b2 = b.reshape(1, V)
    out_t = pl.pallas_call(
        _proj_body_vgrid,
        grid=(_NV,),
        in_specs=[
            pl.BlockSpec((E, B), lambda i: (0, 0)),
            pl.BlockSpec((E, _V_CH), lambda i: (0, i)),
            pl.BlockSpec((1, _V_CH), lambda i: (0, i)),
        ],
        out_specs=pl.BlockSpec((_V_CH, B), lambda i: (i, 0)),
        out_shape=jax.ShapeDtypeStruct((V, B), jnp.float32),
    )(embt, wt, b2)
    return out_t.T


def kernel(data, emb_table, W, b):
    tail = emb_table[_V_ALIGNED:]
    emb = _sc_gather(emb_table.T, data, tail)
    return _tc_project_vgrid(emb.T, W, b)
